# Initial kernel scaffold; baseline (speedup 1.0000x reference)
import jax
import jax.numpy as jnp
from jax.experimental import pallas as pl


def kernel(inputs, targets):
    raise NotImplementedError("write your pallas kernel here")


# speedup vs baseline: 4.8250x; 4.8250x over previous
import functools

import jax
import jax.numpy as jnp
from jax import lax
from jax.experimental import pallas as pl
from jax.experimental.pallas import tpu as pltpu
from jax.experimental.pallas import tpu_sc as plsc

NROW, NCOL = 32768, 128
NSEL = NROW // 2
L = 16
NC, NS = 2, 16
CGL = 4
RGN = 4
ROWS_PER_TILE = NROW // RGN
CH = 512
NCHUNK = ROWS_PER_TILE // CH
BC = 144
BF = 64


def _sc_body(x_hbm, y_hbm, out_hbm, xbuf, ybuf, hist, tmpi, fc, fs, tmpf,
             selb, accb, sh_hist, sh_sel, sh_fc, sh_fs, sh_acc):
    c = lax.axis_index("c")
    s = lax.axis_index("s")
    cgl = lax.rem(s, CGL)
    rg = lax.div(s, CGL)
    g = c * CGL + cgl
    row0 = rg * ROWS_PER_TILE
    lane = lax.iota(jnp.int32, L)
    onesi = jnp.ones((L,), jnp.int32)
    zi = jnp.zeros((L,), jnp.int32)
    zf = jnp.zeros((L,), jnp.float32)

    @pl.loop(0, BC)
    def _(b):
        hist[pl.ds(b * L, L)] = zi

    @pl.loop(0, BF)
    def _(f):
        fc[pl.ds(f * L, L)] = zi
        fs[pl.ds(f * L, L)] = zf

    @pl.loop(0, NCHUNK)
    def _(ch):
        r0 = row0 + ch * CH
        pltpu.sync_copy(x_hbm.at[pl.ds(r0, CH), g, :], xbuf)
        pltpu.sync_copy(y_hbm.at[pl.ds(r0, CH), g, :], ybuf)

        @pl.loop(0, CH, unroll=8)
        def _(i):
            d = jnp.abs(xbuf[i] - ybuf[i])
            v = plsc.bitcast(d, jnp.int32)
            cb = jnp.minimum(jnp.right_shift(v, 23), BC - 1)
            plsc.addupdate_scatter(hist, [cb * L + lane], onesi)

    pltpu.sync_copy(hist, sh_hist.at[s])
    plsc.subcore_barrier()

    @pl.when(rg == 0)
    def _():
        @pl.loop(1, RGN)
        def _(r):
            pltpu.sync_copy(sh_hist.at[r * CGL + cgl], tmpi)

            @pl.loop(0, BC)
            def _(b):
                hist[pl.ds(b * L, L)] = (hist[pl.ds(b * L, L)]
                                         + tmpi[pl.ds(b * L, L)])

        def sel_body(b, carry):
            cum, tv, cbv, found = carry
            h = hist[pl.ds(b * L, L)]
            cum2 = cum + h
            newly = jnp.logical_and(jnp.logical_not(found), cum2 >= NSEL)
            tv = jnp.where(newly, jnp.full((L,), 1, jnp.int32) * b, tv)
            cbv = jnp.where(newly, cum, cbv)
            found = jnp.logical_or(found, newly)
            return cum2, tv, cbv, found

        _, tv, cbv, _ = pl.loop(
            0, BC, init_carry=(zi, zi, zi, zi > 0))(sel_body)
        selb[pl.ds(0, L)] = tv
        selb[pl.ds(L, L)] = cbv
        pltpu.sync_copy(selb, sh_sel.at[cgl])

    plsc.subcore_barrier()

    pltpu.sync_copy(sh_sel.at[cgl], selb)
    tvec = selb[pl.ds(0, L)]

    def p2_chunk(ch, acc):
        r0 = row0 + ch * CH
        pltpu.sync_copy(x_hbm.at[pl.ds(r0, CH), g, :], xbuf)
        pltpu.sync_copy(y_hbm.at[pl.ds(r0, CH), g, :], ybuf)

        def p2_row(i, acc):
            d = jnp.abs(xbuf[i] - ybuf[i])
            v = plsc.bitcast(d, jnp.int32)
            cb = jnp.minimum(jnp.right_shift(v, 23), BC - 1)
            dsq = d * d
            acc = acc + jnp.where(cb < tvec, dsq, zf)
            inb = cb == tvec
            fidx = (jnp.right_shift(v, 17) & (BF - 1)) * L + lane
            plsc.addupdate_scatter(fc, [fidx], onesi, mask=inb)
            plsc.addupdate_scatter(fs, [fidx], dsq, mask=inb)
            return acc

        return pl.loop(0, CH, init_carry=acc, unroll=8)(p2_row)

    acc = pl.loop(0, NCHUNK, init_carry=zf)(p2_chunk)

    accb[...] = acc
    pltpu.sync_copy(accb, sh_acc.at[s])
    pltpu.sync_copy(fc, sh_fc.at[s])
    pltpu.sync_copy(fs, sh_fs.at[s])
    plsc.subcore_barrier()

    @pl.when(rg == 0)
    def _():
        cbv = selb[pl.ds(L, L)]

        @pl.loop(1, RGN)
        def _(r):
            sid = r * CGL + cgl
            pltpu.sync_copy(sh_fc.at[sid], tmpi.at[pl.ds(0, BF * L)])
            pltpu.sync_copy(sh_fs.at[sid], tmpf)

            @pl.loop(0, BF)
            def _(f):
                fc[pl.ds(f * L, L)] = (fc[pl.ds(f * L, L)]
                                       + tmpi[pl.ds(f * L, L)])
                fs[pl.ds(f * L, L)] = (fs[pl.ds(f * L, L)]
                                       + tmpf[pl.ds(f * L, L)])

        def acc_body(r, a):
            pltpu.sync_copy(sh_acc.at[r * CGL + cgl], accb)
            return a + accb[...]

        below = pl.loop(0, RGN, init_carry=zf)(acc_body)

        rv = NSEL - cbv

        def fin_body(f, carry):
            cumc, cums, res, found = carry
            fcb = fc[pl.ds(f * L, L)]
            fsb = fs[pl.ds(f * L, L)]
            cumc2 = cumc + fcb
            newly = jnp.logical_and(jnp.logical_not(found), cumc2 >= rv)
            part = cums + ((rv - cumc).astype(jnp.float32) * fsb
                           / jnp.maximum(fcb, 1).astype(jnp.float32))
            res = jnp.where(newly, part, res)
            found = jnp.logical_or(found, newly)
            return cumc2, cums + fsb, res, found

        _, _, fin, _ = pl.loop(
            0, BF, init_carry=(zi, zf, zf, zi > 0))(fin_body)

        accb[...] = below + fin
        pltpu.sync_copy(accb, out_hbm.at[pl.ds(g * L, L)])


def _make_sc_kernel():
    mesh = plsc.VectorSubcoreMesh(
        core_axis_name="c", subcore_axis_name="s", num_cores=NC,
        num_subcores=NS)
    scratch = [
        pltpu.VMEM((CH, L), jnp.float32),
        pltpu.VMEM((CH, L), jnp.float32),
        pltpu.VMEM((BC * L,), jnp.int32),
        pltpu.VMEM((BC * L,), jnp.int32),
        pltpu.VMEM((BF * L,), jnp.int32),
        pltpu.VMEM((BF * L,), jnp.float32),
        pltpu.VMEM((BF * L,), jnp.float32),
        pltpu.VMEM((2 * L,), jnp.int32),
        pltpu.VMEM((L,), jnp.float32),
        pltpu.VMEM_SHARED((NS, BC * L), jnp.int32),
        pltpu.VMEM_SHARED((CGL, 2 * L), jnp.int32),
        pltpu.VMEM_SHARED((NS, BF * L), jnp.int32),
        pltpu.VMEM_SHARED((NS, BF * L), jnp.float32),
        pltpu.VMEM_SHARED((NS, L), jnp.float32),
    ]

    return pl.kernel(
        _sc_body,
        out_type=jax.ShapeDtypeStruct((NCOL,), jnp.float32),
        mesh=mesh,
        scratch_types=scratch,
        compiler_params=pltpu.CompilerParams(
            needs_layout_passes=False, use_tc_tiling_on_sc=False),
    )


_sc_call = _make_sc_kernel()


@jax.jit
def kernel(inputs, targets):
    x3 = inputs.reshape(NROW, NCOL // L, L)
    y3 = targets.reshape(NROW, NCOL // L, L)
    colsums = _sc_call(x3, y3)
    return jnp.sum(colsums) / jnp.float32(NSEL * NCOL)


# speedup vs baseline: 11.0559x; 2.2914x over previous
import jax
import jax.numpy as jnp
from jax import lax
from jax.experimental import pallas as pl
from jax.experimental.pallas import tpu as pltpu
from jax.experimental.pallas import tpu_sc as plsc

NROW, NCOL = 32768, 128
NSEL = NROW // 2
L = 16
NC, NS = 2, 16
CGL = 4
RGN = 4
ROWS_PER_TILE = NROW // RGN
CH = 512
NCHUNK = ROWS_PER_TILE // CH
BC = 144
BF = 64
RC = 4
RF = 2


def _sc_body(x_hbm, y_hbm, out_hbm, xbuf, ybuf, hist, tmpi, fc, fs, tmpf,
             selb, accb, semx0, semy0, semx1, semy1,
             sh_hist, sh_sel, sh_fc, sh_fs, sh_acc):
    c = lax.axis_index("c")
    s = lax.axis_index("s")
    cgl = lax.rem(s, CGL)
    rg = lax.div(s, CGL)
    g = c * CGL + cgl
    col0 = g * L
    row0 = rg * ROWS_PER_TILE
    lane = lax.iota(jnp.int32, L)
    onesi = jnp.ones((L,), jnp.int32)
    zi = jnp.zeros((L,), jnp.int32)
    zf = jnp.zeros((L,), jnp.float32)
    sems = ((semx0, semy0), (semx1, semy1))

    @pl.loop(0, RC * BC)
    def _(b):
        hist[pl.ds(b * L, L)] = zi

    @pl.loop(0, RF * BF)
    def _(f):
        fc[pl.ds(f * L, L)] = zi
        fs[pl.ds(f * L, L)] = zf

    def issue(ch, slot):
        r0 = row0 + ch * CH
        hx = pltpu.async_copy(
            x_hbm.at[pl.ds(r0, CH), pl.ds(col0, L)],
            xbuf.at[pl.ds(slot * CH, CH), :], sems[slot][0])
        hy = pltpu.async_copy(
            y_hbm.at[pl.ds(r0, CH), pl.ds(col0, L)],
            ybuf.at[pl.ds(slot * CH, CH), :], sems[slot][1])
        return hx, hy

    pending = {0: issue(0, 0)}
    for ch in range(NCHUNK):
        slot = ch % 2
        if ch + 1 < NCHUNK:
            pending[(ch + 1) % 2] = issue(ch + 1, (ch + 1) % 2)
        hx, hy = pending[slot]
        hx.wait()
        hy.wait()
        base = slot * CH

        @pl.loop(0, CH, step=8)
        def _(i):
            for k in range(8):
                xv = xbuf[base + i + k]
                yv = ybuf[base + i + k]
                d = jnp.abs(xv - yv)
                v = plsc.bitcast(d, jnp.int32)
                cb = jnp.minimum(jnp.right_shift(v, 23), BC - 1)
                rep = (k % RC) * (BC * L)
                plsc.addupdate_scatter(hist, [cb * L + lane + rep], onesi)

    @pl.loop(0, BC)
    def _(b):
        tot = hist[pl.ds(b * L, L)]
        for r in range(1, RC):
            tot = tot + hist[pl.ds(r * BC * L + b * L, L)]
        hist[pl.ds(b * L, L)] = tot

    pltpu.sync_copy(hist.at[pl.ds(0, BC * L)], sh_hist.at[s])
    plsc.subcore_barrier()

    @pl.when(rg == 0)
    def _():
        @pl.loop(1, RGN)
        def _(r):
            pltpu.sync_copy(sh_hist.at[r * CGL + cgl], tmpi)

            @pl.loop(0, BC)
            def _(b):
                hist[pl.ds(b * L, L)] = (hist[pl.ds(b * L, L)]
                                         + tmpi[pl.ds(b * L, L)])

        def sel_body(b, carry):
            cum, tv, cbv, found = carry
            h = hist[pl.ds(b * L, L)]
            cum2 = cum + h
            newly = jnp.logical_and(jnp.logical_not(found), cum2 >= NSEL)
            tv = jnp.where(newly, jnp.full((L,), 1, jnp.int32) * b, tv)
            cbv = jnp.where(newly, cum, cbv)
            found = jnp.logical_or(found, newly)
            return cum2, tv, cbv, found

        _, tv, cbv, _ = pl.loop(
            0, BC, init_carry=(zi, zi, zi, zi > 0))(sel_body)
        selb[pl.ds(0, L)] = tv
        selb[pl.ds(L, L)] = cbv
        pltpu.sync_copy(selb, sh_sel.at[cgl])

    plsc.subcore_barrier()

    pltpu.sync_copy(sh_sel.at[cgl], selb)
    tvec = selb[pl.ds(0, L)]

    pending = {0: issue(0, 0)}
    acc = zf
    for ch in range(NCHUNK):
        slot = ch % 2
        if ch + 1 < NCHUNK:
            pending[(ch + 1) % 2] = issue(ch + 1, (ch + 1) % 2)
        hx, hy = pending[slot]
        hx.wait()
        hy.wait()
        base = slot * CH

        def p2_block(i, acc):
            for k in range(8):
                xv = xbuf[base + i + k]
                yv = ybuf[base + i + k]
                d = jnp.abs(xv - yv)
                v = plsc.bitcast(d, jnp.int32)
                cb = jnp.right_shift(v, 23)
                dsq = d * d
                acc = acc + jnp.where(cb < tvec, dsq, zf)
                inb = cb == tvec
                rep = (k % RF) * (BF * L)
                fidx = (jnp.right_shift(v, 17) & (BF - 1)) * L + lane + rep
                plsc.addupdate_scatter(fc, [fidx], onesi, mask=inb)
                plsc.addupdate_scatter(fs, [fidx], dsq, mask=inb)
            return acc

        acc = pl.loop(0, CH, step=8, init_carry=acc)(p2_block)

    @pl.loop(0, BF)
    def _(f):
        ctot = fc[pl.ds(f * L, L)]
        stot = fs[pl.ds(f * L, L)]
        for r in range(1, RF):
            ctot = ctot + fc[pl.ds(r * BF * L + f * L, L)]
            stot = stot + fs[pl.ds(r * BF * L + f * L, L)]
        fc[pl.ds(f * L, L)] = ctot
        fs[pl.ds(f * L, L)] = stot

    accb[...] = acc
    pltpu.sync_copy(accb, sh_acc.at[s])
    pltpu.sync_copy(fc.at[pl.ds(0, BF * L)], sh_fc.at[s])
    pltpu.sync_copy(fs.at[pl.ds(0, BF * L)], sh_fs.at[s])
    plsc.subcore_barrier()

    @pl.when(rg == 0)
    def _():
        cbv = selb[pl.ds(L, L)]

        @pl.loop(1, RGN)
        def _(r):
            sid = r * CGL + cgl
            pltpu.sync_copy(sh_fc.at[sid], tmpi.at[pl.ds(0, BF * L)])
            pltpu.sync_copy(sh_fs.at[sid], tmpf)

            @pl.loop(0, BF)
            def _(f):
                fc[pl.ds(f * L, L)] = (fc[pl.ds(f * L, L)]
                                       + tmpi[pl.ds(f * L, L)])
                fs[pl.ds(f * L, L)] = (fs[pl.ds(f * L, L)]
                                       + tmpf[pl.ds(f * L, L)])

        def acc_body(r, a):
            pltpu.sync_copy(sh_acc.at[r * CGL + cgl], accb)
            return a + accb[...]

        below = pl.loop(0, RGN, init_carry=zf)(acc_body)

        rv = NSEL - cbv

        def fin_body(f, carry):
            cumc, cums, res, found = carry
            fcb = fc[pl.ds(f * L, L)]
            fsb = fs[pl.ds(f * L, L)]
            cumc2 = cumc + fcb
            newly = jnp.logical_and(jnp.logical_not(found), cumc2 >= rv)
            part = cums + ((rv - cumc).astype(jnp.float32) * fsb
                           / jnp.maximum(fcb, 1).astype(jnp.float32))
            res = jnp.where(newly, part, res)
            found = jnp.logical_or(found, newly)
            return cumc2, cums + fsb, res, found

        _, _, fin, _ = pl.loop(
            0, BF, init_carry=(zi, zf, zf, zi > 0))(fin_body)

        accb[...] = below + fin
        pltpu.sync_copy(accb, out_hbm.at[pl.ds(g * L, L)])


def _make_sc_kernel():
    mesh = plsc.VectorSubcoreMesh(
        core_axis_name="c", subcore_axis_name="s", num_cores=NC,
        num_subcores=NS)
    scratch = [
        pltpu.VMEM((2 * CH, L), jnp.float32),
        pltpu.VMEM((2 * CH, L), jnp.float32),
        pltpu.VMEM((RC * BC * L,), jnp.int32),
        pltpu.VMEM((BC * L,), jnp.int32),
        pltpu.VMEM((RF * BF * L,), jnp.int32),
        pltpu.VMEM((RF * BF * L,), jnp.float32),
        pltpu.VMEM((BF * L,), jnp.float32),
        pltpu.VMEM((2 * L,), jnp.int32),
        pltpu.VMEM((L,), jnp.float32),
        pltpu.SemaphoreType.DMA,
        pltpu.SemaphoreType.DMA,
        pltpu.SemaphoreType.DMA,
        pltpu.SemaphoreType.DMA,
        pltpu.VMEM_SHARED((NS, BC * L), jnp.int32),
        pltpu.VMEM_SHARED((CGL, 2 * L), jnp.int32),
        pltpu.VMEM_SHARED((NS, BF * L), jnp.int32),
        pltpu.VMEM_SHARED((NS, BF * L), jnp.float32),
        pltpu.VMEM_SHARED((NS, L), jnp.float32),
    ]

    return pl.kernel(
        _sc_body,
        out_type=jax.ShapeDtypeStruct((NCOL,), jnp.float32),
        mesh=mesh,
        scratch_types=scratch,
        compiler_params=pltpu.CompilerParams(
            needs_layout_passes=False, use_tc_tiling_on_sc=False),
    )


_sc_call = _make_sc_kernel()


@jax.jit
def kernel(inputs, targets):
    colsums = _sc_call(inputs, targets)
    return jnp.sum(colsums) / jnp.float32(NSEL * NCOL)


# speedup vs baseline: 19.3907x; 1.7539x over previous
import jax
import jax.numpy as jnp
from jax import lax
from jax.experimental import pallas as pl
from jax.experimental.pallas import tpu as pltpu
from jax.experimental.pallas import tpu_sc as plsc

NROW, NCOL = 32768, 128
NSEL = NROW // 2
L = 16
NC, NS = 2, 16
CGL = 4
RGN = 4
ROWS_PER_TILE = NROW // RGN
CH = 512
NCHUNK = ROWS_PER_TILE // CH
NB = 1024
SHIFT = 21
MW = (NB // RGN) * L


def _sc_body(x_hbm, y_hbm, out_hbm, xbuf, ybuf, cnt, summ, accb,
             semx0, semy0, semx1, semy1, sh_cnt, sh_sum):
    c = lax.axis_index("c")
    s = lax.axis_index("s")
    cgl = lax.rem(s, CGL)
    rg = lax.div(s, CGL)
    g = c * CGL + cgl
    col0 = g * L
    row0 = rg * ROWS_PER_TILE
    lane = lax.iota(jnp.int32, L)
    onesi = jnp.ones((L,), jnp.int32)
    zi = jnp.zeros((L,), jnp.int32)
    zf = jnp.zeros((L,), jnp.float32)
    sems = ((semx0, semy0), (semx1, semy1))

    @pl.loop(0, NB)
    def _(b):
        cnt[pl.ds(b * L, L)] = zi
        summ[pl.ds(b * L, L)] = zf

    def issue(ch, slot):
        r0 = row0 + ch * CH
        hx = pltpu.async_copy(
            x_hbm.at[pl.ds(r0, CH), pl.ds(col0, L)],
            xbuf.at[pl.ds(slot * CH, CH), :], sems[slot][0])
        hy = pltpu.async_copy(
            y_hbm.at[pl.ds(r0, CH), pl.ds(col0, L)],
            ybuf.at[pl.ds(slot * CH, CH), :], sems[slot][1])
        return hx, hy

    pending = {0: issue(0, 0)}
    for ch in range(NCHUNK):
        slot = ch % 2
        if ch + 1 < NCHUNK:
            pending[(ch + 1) % 2] = issue(ch + 1, (ch + 1) % 2)
        hx, hy = pending[slot]
        hx.wait()
        hy.wait()
        base = slot * CH

        @pl.loop(0, CH, step=8)
        def _(i):
            for k in range(8):
                xv = xbuf[base + i + k]
                yv = ybuf[base + i + k]
                d = xv - yv
                v = plsc.bitcast(d, jnp.int32) & jnp.int32(0x7FFFFFFF)
                idx = (jnp.right_shift(v, SHIFT - 4)
                       & jnp.int32((NB - 1) * L)) | lane
                plsc.addupdate_scatter(cnt, [idx], onesi)
                plsc.addupdate_scatter(summ, [idx], d * d)

    pltpu.sync_copy(cnt, sh_cnt.at[s])
    pltpu.sync_copy(summ, sh_sum.at[s])
    plsc.subcore_barrier()

    off = rg * MW
    pltpu.sync_copy(sh_cnt.at[cgl, pl.ds(off, MW)], cnt.at[pl.ds(0, MW)])
    pltpu.sync_copy(sh_sum.at[cgl, pl.ds(off, MW)], summ.at[pl.ds(0, MW)])
    for r in range(1, RGN):
        sid = r * CGL + cgl
        pltpu.sync_copy(sh_cnt.at[sid, pl.ds(off, MW)],
                        cnt.at[pl.ds(MW, MW)])
        pltpu.sync_copy(sh_sum.at[sid, pl.ds(off, MW)],
                        summ.at[pl.ds(MW, MW)])

        @pl.loop(0, MW // L)
        def _(b):
            cnt[pl.ds(b * L, L)] = (cnt[pl.ds(b * L, L)]
                                    + cnt[pl.ds(MW + b * L, L)])
            summ[pl.ds(b * L, L)] = (summ[pl.ds(b * L, L)]
                                     + summ[pl.ds(MW + b * L, L)])

    pltpu.sync_copy(cnt.at[pl.ds(0, MW)], sh_cnt.at[cgl, pl.ds(off, MW)])
    pltpu.sync_copy(summ.at[pl.ds(0, MW)], sh_sum.at[cgl, pl.ds(off, MW)])
    plsc.subcore_barrier()

    @pl.when(rg == 0)
    def _():
        pltpu.sync_copy(sh_cnt.at[cgl], cnt)
        pltpu.sync_copy(sh_sum.at[cgl], summ)

        def scan_body(b, carry):
            cum, cums, res, found = carry
            cb = cnt[pl.ds(b * L, L)]
            sb = summ[pl.ds(b * L, L)]
            cum2 = cum + cb
            newly = jnp.logical_and(jnp.logical_not(found), cum2 >= NSEL)
            edge = plsc.bitcast(onesi * lax.shift_left(b, SHIFT),
                                jnp.float32)
            s2 = edge * edge
            rvf = (NSEL - cum).astype(jnp.float32)
            cf = jnp.maximum(cb, 1).astype(jnp.float32)
            est = rvf * s2 + rvf * rvf / cf * (sb / cf - s2)
            res = jnp.where(newly, cums + est, res)
            found = jnp.logical_or(found, newly)
            return cum2, cums + sb, res, found

        _, _, fin, _ = pl.loop(
            0, NB, init_carry=(zi, zf, zf, zi > 0))(scan_body)

        accb[...] = fin
        pltpu.sync_copy(accb, out_hbm.at[pl.ds(g * L, L)])


def _make_sc_kernel():
    mesh = plsc.VectorSubcoreMesh(
        core_axis_name="c", subcore_axis_name="s", num_cores=NC,
        num_subcores=NS)
    scratch = [
        pltpu.VMEM((2 * CH, L), jnp.float32),
        pltpu.VMEM((2 * CH, L), jnp.float32),
        pltpu.VMEM((NB * L,), jnp.int32),
        pltpu.VMEM((NB * L,), jnp.float32),
        pltpu.VMEM((L,), jnp.float32),
        pltpu.SemaphoreType.DMA,
        pltpu.SemaphoreType.DMA,
        pltpu.SemaphoreType.DMA,
        pltpu.SemaphoreType.DMA,
        pltpu.VMEM_SHARED((NS, NB * L), jnp.int32),
        pltpu.VMEM_SHARED((NS, NB * L), jnp.float32),
    ]

    return pl.kernel(
        _sc_body,
        out_type=jax.ShapeDtypeStruct((NCOL,), jnp.float32),
        mesh=mesh,
        scratch_types=scratch,
        compiler_params=pltpu.CompilerParams(
            needs_layout_passes=False, use_tc_tiling_on_sc=False),
    )


_sc_call = _make_sc_kernel()


@jax.jit
def kernel(inputs, targets):
    colsums = _sc_call(inputs, targets)
    return jnp.sum(colsums) / jnp.float32(NSEL * NCOL)


# speedup vs baseline: 36.1717x; 1.8654x over previous
import jax
import jax.numpy as jnp
from jax import lax
from jax.experimental import pallas as pl
from jax.experimental.pallas import tpu as pltpu
from jax.experimental.pallas import tpu_sc as plsc

NROW, NCOL = 32768, 128
NSEL = NROW // 2
L = 16
NC, NS = 2, 16
CGL = 4
RGN = 4
ROWS_PER_TILE = NROW // RGN
CH = 512
NCHUNK = ROWS_PER_TILE // CH
NB = 1024
SHIFT = 21
MW = (NB // RGN) * L


def _sc_body(x_hbm, y_hbm, out_hbm, xbuf, ybuf, cnt, summ, accb,
             semx0, semy0, semx1, semy1, sh_cnt, sh_sum):
    c = lax.axis_index("c")
    s = lax.axis_index("s")
    cgl = lax.rem(s, CGL)
    rg = lax.div(s, CGL)
    g = c * CGL + cgl
    col0 = g * L
    row0 = rg * ROWS_PER_TILE
    lane = lax.iota(jnp.int32, L)
    onesi = jnp.ones((L,), jnp.int32)
    zi = jnp.zeros((L,), jnp.int32)
    zf = jnp.zeros((L,), jnp.float32)
    sems = ((semx0, semy0), (semx1, semy1))

    @pl.loop(0, NB)
    def _(b):
        cnt[pl.ds(b * L, L)] = zi
        summ[pl.ds(b * L, L)] = zf

    def issue(ch, slot):
        r0 = row0 + ch * CH
        hx = pltpu.async_copy(
            x_hbm.at[pl.ds(r0, CH), pl.ds(col0, L)],
            xbuf.at[pl.ds(slot * CH, CH), :], sems[slot][0])
        hy = pltpu.async_copy(
            y_hbm.at[pl.ds(r0, CH), pl.ds(col0, L)],
            ybuf.at[pl.ds(slot * CH, CH), :], sems[slot][1])
        return hx, hy

    pending = {0: issue(0, 0)}
    for ch in range(NCHUNK):
        slot = ch % 2
        if ch + 1 < NCHUNK:
            pending[(ch + 1) % 2] = issue(ch + 1, (ch + 1) % 2)
        hx, hy = pending[slot]
        hx.wait()
        hy.wait()
        base = slot * CH

        @plsc.parallel_loop(0, CH, unroll=8)
        def _(i):
            xv = xbuf[base + i]
            yv = ybuf[base + i]
            d = xv - yv
            v = plsc.bitcast(d, jnp.int32)
            idx = (lax.shift_right_logical(v, SHIFT - 4)
                   & jnp.int32((NB - 1) * L)) | lane
            plsc.addupdate_scatter(cnt, [idx], onesi)
            plsc.addupdate_scatter(summ, [idx], d * d)

    pltpu.sync_copy(cnt, sh_cnt.at[s])
    pltpu.sync_copy(summ, sh_sum.at[s])
    plsc.subcore_barrier()

    off = rg * MW
    pltpu.sync_copy(sh_cnt.at[cgl, pl.ds(off, MW)], cnt.at[pl.ds(0, MW)])
    pltpu.sync_copy(sh_sum.at[cgl, pl.ds(off, MW)], summ.at[pl.ds(0, MW)])
    for r in range(1, RGN):
        sid = r * CGL + cgl
        pltpu.sync_copy(sh_cnt.at[sid, pl.ds(off, MW)],
                        cnt.at[pl.ds(MW, MW)])
        pltpu.sync_copy(sh_sum.at[sid, pl.ds(off, MW)],
                        summ.at[pl.ds(MW, MW)])

        @pl.loop(0, MW // L)
        def _(b):
            cnt[pl.ds(b * L, L)] = (cnt[pl.ds(b * L, L)]
                                    + cnt[pl.ds(MW + b * L, L)])
            summ[pl.ds(b * L, L)] = (summ[pl.ds(b * L, L)]
                                     + summ[pl.ds(MW + b * L, L)])

    pltpu.sync_copy(cnt.at[pl.ds(0, MW)], sh_cnt.at[cgl, pl.ds(off, MW)])
    pltpu.sync_copy(summ.at[pl.ds(0, MW)], sh_sum.at[cgl, pl.ds(off, MW)])
    plsc.subcore_barrier()

    @pl.when(rg == 0)
    def _():
        pltpu.sync_copy(sh_cnt.at[cgl], cnt)
        pltpu.sync_copy(sh_sum.at[cgl], summ)

        def scan_body(b, carry):
            cum, cums, res, found = carry
            cb = cnt[pl.ds(b * L, L)]
            sb = summ[pl.ds(b * L, L)]
            cum2 = cum + cb
            newly = jnp.logical_and(jnp.logical_not(found), cum2 >= NSEL)
            edge = plsc.bitcast(onesi * lax.shift_left(b, SHIFT),
                                jnp.float32)
            s2 = edge * edge
            rvf = (NSEL - cum).astype(jnp.float32)
            cf = jnp.maximum(cb, 1).astype(jnp.float32)
            est = rvf * s2 + rvf * rvf / cf * (sb / cf - s2)
            res = jnp.where(newly, cums + est, res)
            found = jnp.logical_or(found, newly)
            return cum2, cums + sb, res, found

        _, _, fin, _ = pl.loop(
            0, NB, init_carry=(zi, zf, zf, zi > 0))(scan_body)

        accb[...] = fin
        pltpu.sync_copy(accb, out_hbm.at[pl.ds(g * L, L)])


def _make_sc_kernel():
    mesh = plsc.VectorSubcoreMesh(
        core_axis_name="c", subcore_axis_name="s", num_cores=NC,
        num_subcores=NS)
    scratch = [
        pltpu.VMEM((2 * CH, L), jnp.float32),
        pltpu.VMEM((2 * CH, L), jnp.float32),
        pltpu.VMEM((NB * L,), jnp.int32),
        pltpu.VMEM((NB * L,), jnp.float32),
        pltpu.VMEM((L,), jnp.float32),
        pltpu.SemaphoreType.DMA,
        pltpu.SemaphoreType.DMA,
        pltpu.SemaphoreType.DMA,
        pltpu.SemaphoreType.DMA,
        pltpu.VMEM_SHARED((NS, NB * L), jnp.int32),
        pltpu.VMEM_SHARED((NS, NB * L), jnp.float32),
    ]

    return pl.kernel(
        _sc_body,
        out_type=jax.ShapeDtypeStruct((NCOL,), jnp.float32),
        mesh=mesh,
        scratch_types=scratch,
        compiler_params=pltpu.CompilerParams(
            needs_layout_passes=False, use_tc_tiling_on_sc=False),
    )


_sc_call = _make_sc_kernel()


@jax.jit
def kernel(inputs, targets):
    colsums = _sc_call(inputs, targets)
    return jnp.sum(colsums) / jnp.float32(NSEL * NCOL)
